# baseline (speedup 1.0000x reference)
import functools

import jax
import jax.numpy as jnp
from jax import lax
from jax.experimental import pallas as pl
from jax.experimental.pallas import tpu as pltpu
from jax.experimental.pallas import tpu_sc as plsc

_N = 10000
_E = 320000
_NC = 2
_NS = 16
_NW = _NC * _NS
_EW = _E // _NW
_C = 80
_NCH = _EW // _C
_G = _C // 16
_RPT = _N // _NS
_BN = 1000


def _mesh():
    return plsc.VectorSubcoreMesh(
        core_axis_name="c", subcore_axis_name="s",
        num_cores=_NC, num_subcores=_NS)


_DS = 64


def _make_sc_layer(NP):
    @functools.partial(
        pl.kernel,
        out_type=(
            jax.ShapeDtypeStruct((NP, _NC, _N, _DS), jnp.float32),
            jax.ShapeDtypeStruct((_NC, _N, 16), jnp.float32),
        ),
        mesh=_mesh(),
        compiler_params=pltpu.CompilerParams(
            needs_layout_passes=False, use_tc_tiling_on_sc=False),
        scratch_types=[
            pltpu.VMEM((_NCH, _C), jnp.int32),
            pltpu.VMEM((_NCH, _C), jnp.int32),
            pltpu.VMEM((_N,), jnp.float32),
            pltpu.VMEM((_N,), jnp.float32),
            pltpu.VMEM((_NCH, _C), jnp.float32),
        ] + [pltpu.VMEM((_C, _DS), jnp.bfloat16)] * 4
          + [pltpu.VMEM((_C, _DS), jnp.float32)] * 2
          + [pltpu.VMEM((_C, 16), jnp.float32)] * 2
          + [
            pltpu.VMEM_SHARED((_N, _DS), jnp.float32),
            pltpu.VMEM_SHARED((_N, 16), jnp.float32),
        ] + [pltpu.SemaphoreType.DMA] * 12,
    )
    def sc_layer(*refs):
        h_hbms = refs[:NP]
        (el_hbm, er_hbm, src_hbm, dst_hbm, u_hbm, t_hbm,
         src_v, dst_v, el_v, er_v, ee_v) = refs[NP:NP + 11]
        hbf_bufs = list(refs[NP + 11:NP + 15])
        rows_bufs = list(refs[NP + 15:NP + 17])
        eec_bufs = list(refs[NP + 17:NP + 19])
        acc_sh, den_sh = refs[NP + 19:NP + 21]
        sems = refs[NP + 21:]
        sg = sems[0:4]
        ss = sems[4:6]
        se = sems[6:8]
        rows_v, eec_v = rows_bufs[0], eec_bufs[0]
        cid = lax.axis_index("c")
        sid = lax.axis_index("s")
        wid = sid * _NC + cid

        pltpu.sync_copy(src_hbm.at[wid], src_v)
        pltpu.sync_copy(dst_hbm.at[wid], dst_v)
        pltpu.sync_copy(el_hbm, el_v)
        pltpu.sync_copy(er_hbm, er_v)

        zero16 = jnp.zeros((16,), jnp.float32)

        def zero_rows(i, _):
            r = i // (_DS // 16)
            g = i - r * (_DS // 16)
            rows_v[r, pl.ds(pl.multiple_of(g * 16, 16), 16)] = zero16
            return 0
        lax.fori_loop(0, _C * _DS // 16, zero_rows, 0)

        def zero_eec(r, _):
            eec_v[r, :] = zero16
            return 0
        lax.fori_loop(0, _C, zero_eec, 0)

        def owned_chunks(fn):
            for k in range(-(-_NCH // _NS)):
                c = sid + _NS * k

                @pl.when(c < _NCH)
                def _run(c=c):
                    fn(pl.ds(pl.multiple_of(c * _C, _C), _C))

        def zero_acc(off):
            pltpu.sync_copy(rows_v, acc_sh.at[off])

        owned_chunks(zero_acc)
        owned_chunks(lambda off: pltpu.sync_copy(eec_v, den_sh.at[off]))
        plsc.subcore_barrier()

        def edge_body(c, _):
            for g in range(_G):
                off = pl.ds(g * 16, 16)
                s16 = src_v[c, off]
                d16 = dst_v[c, off]
                el16 = plsc.load_gather(el_v, [s16])
                er16 = plsc.load_gather(er_v, [d16])
                e = el16 + er16
                e = jnp.where(e > 0.0, e, 0.2 * e)
                ee_v[c, off] = jnp.exp(e)
            return 0
        lax.fori_loop(0, _NCH, edge_body, 0)

        lane0 = lax.iota(jnp.int32, 16) == 0
        _ROUNDS = _NCH // 4

        for p in range(NP):
            h_hbm = h_hbms[p]

            def scale(hbf, rows, eec, c, p=p):
                fc = jnp.full((16,), c, jnp.int32)
                himask = jnp.full((16,), 0xFFFF0000, jnp.uint32)

                def scale_rows4(q, _):
                    r0 = q * 4
                    for j in range(4):
                        r = r0 + j
                        b = plsc.load_gather(
                            ee_v, [fc, jnp.full((16,), r, jnp.int32)])
                        if p == 0:
                            eec[r, :] = jnp.where(lane0, b, 0.0)
                        for dg in range(_DS // 32):
                            v = hbf[r, pl.ds(dg * 32, 32)]
                            u = plsc.bitcast(v, jnp.uint32)
                            lo = plsc.bitcast(u << 16, jnp.float32)
                            hi = plsc.bitcast(u & himask, jnp.float32)
                            rows[r, pl.ds(dg * 32, 16)] = lo * b
                            rows[r, pl.ds(dg * 32 + 16, 16)] = hi * b
                    return 0
                lax.fori_loop(0, _C // 4, scale_rows4, 0)

            def gather_start(c, hbf, sgj, h_hbm=h_hbm):
                pltpu.async_copy(h_hbm.at[src_v.at[c]], hbf, sgj)

            def gather_wait(c, hbf, sgj, h_hbm=h_hbm):
                pltpu.make_async_copy(h_hbm.at[src_v.at[c]], hbf, sgj).wait()

            def scatter_start(c, rows, eec, ss, se, p=p):
                pltpu.async_copy(rows, acc_sh.at[dst_v.at[c]], ss, add=True)
                if p == 0:
                    pltpu.async_copy(eec, den_sh.at[dst_v.at[c]], se,
                                     add=True)

            def scatter_wait(c, rows, eec, ss, se, p=p):
                pltpu.make_async_copy(rows, acc_sh.at[dst_v.at[c]], ss).wait()
                if p == 0:
                    pltpu.make_async_copy(eec, den_sh.at[dst_v.at[c]],
                                          se).wait()

            for j in range(3):
                gather_start(j, hbf_bufs[j], sg[j])

            def ring_body(k, _):
                for j in range(4):
                    c = 4 * k + j
                    jr = j % 2
                    jp3 = (j + 3) % 4
                    gather_wait(c, hbf_bufs[j], sg[j])

                    @pl.when(c + 3 < _NCH)
                    def _next_gather():
                        gather_start(c + 3, hbf_bufs[jp3], sg[jp3])
                    if j < 2:
                        @pl.when(k > 0)
                        def _wait_prev():
                            scatter_wait(c - 2, rows_bufs[jr],
                                         eec_bufs[jr], ss[jr], se[jr])
                    else:
                        scatter_wait(c - 2, rows_bufs[jr], eec_bufs[jr],
                                     ss[jr], se[jr])
                    scale(hbf_bufs[j], rows_bufs[jr], eec_bufs[jr], c)
                    scatter_start(c, rows_bufs[jr], eec_bufs[jr],
                                  ss[jr], se[jr])
                return 0
            lax.fori_loop(0, _ROUNDS, ring_body, 0)

            last = _NCH - 1
            jl = last % 4
            jlr = last % 2
            gather_wait(last, hbf_bufs[jl], sg[jl])
            scatter_wait(last - 2, rows_bufs[jlr], eec_bufs[jlr],
                         ss[jlr], se[jlr])
            scale(hbf_bufs[jl], rows_bufs[jlr], eec_bufs[jlr], last)
            scatter_start(last, rows_bufs[jlr], eec_bufs[jlr],
                          ss[jlr], se[jlr])
            for d in range(last - 1, last + 1):
                jd = d % 2
                scatter_wait(d, rows_bufs[jd], eec_bufs[jd], ss[jd], se[jd])

            plsc.subcore_barrier()

            owned_chunks(
                lambda off, p=p: pltpu.sync_copy(acc_sh.at[off],
                                                 u_hbm.at[p, cid, off]))
            if p == 0:
                owned_chunks(
                    lambda off: pltpu.sync_copy(den_sh.at[off],
                                                t_hbm.at[cid, off]))
            if p + 1 < NP:
                lax.fori_loop(0, _C * _DS // 16, zero_rows, 0)
                owned_chunks(zero_acc)
                plsc.subcore_barrier()

    return sc_layer


_sc_layer1 = _make_sc_layer(2)
_sc_layer2 = _make_sc_layer(1)


def _h_out_specs(dout):
    np_ = dout // _DS
    specs = [pl.BlockSpec((_BN, _DS), lambda i: (i, 0))] * np_
    specs += [pl.BlockSpec((_BN, 1), lambda i: (i, 0))] * 2
    shapes = [jax.ShapeDtypeStruct((_N, _DS), jnp.bfloat16)] * np_
    shapes += [jax.ShapeDtypeStruct((_N, 1), jnp.float32)] * 2
    return specs, shapes


def _write_h(h, al_ref, ar_ref, out_refs):
    np_ = len(out_refs) - 2
    for p in range(np_):
        part = h[:, p * _DS:(p + 1) * _DS].reshape(-1, _DS // 32, 2, 16)
        inter = jnp.stack([part[:, :, 0, :], part[:, :, 1, :]], axis=-1)
        out_refs[p][...] = inter.reshape(-1, _DS).astype(jnp.bfloat16)
    out_refs[np_][...] = jnp.dot(h, al_ref[...],
                                 preferred_element_type=jnp.float32)[:, None]
    out_refs[np_ + 1][...] = jnp.dot(h, ar_ref[...],
                                     preferred_element_type=jnp.float32)[:, None]


def _tc_project_body(x_ref, w_ref, al_ref, ar_ref, *out_refs):
    h = jnp.dot(x_ref[...], w_ref[...], preferred_element_type=jnp.float32)
    _write_h(h, al_ref, ar_ref, out_refs)


def _tc_project(x, w, al, ar):
    din, dout = w.shape
    out_specs, out_shape = _h_out_specs(dout)
    return pl.pallas_call(
        _tc_project_body,
        grid=(_N // _BN,),
        in_specs=[
            pl.BlockSpec((_BN, din), lambda i: (i, 0)),
            pl.BlockSpec((din, dout), lambda i: (0, 0)),
            pl.BlockSpec((dout,), lambda i: (0,)),
            pl.BlockSpec((dout,), lambda i: (0,)),
        ],
        out_specs=out_specs,
        out_shape=out_shape,
    )(x, w, al, ar)


def _norm(u_ref, t_ref, b_ref):
    np_ = u_ref.shape[0]
    u = jnp.concatenate(
        [u_ref[p, 0] + u_ref[p, 1] for p in range(np_)], axis=1)
    s = t_ref[0, :, 0:1] + t_ref[1, :, 0:1]
    s = jnp.where(s == 0.0, 1.0, s)
    return u / s + b_ref[...][None, :]


def _tc_mid_body(u_ref, t_ref, b_ref, w_ref, al_ref, ar_ref, *out_refs):
    x = jnp.maximum(_norm(u_ref, t_ref, b_ref), 0.0)
    h = jnp.dot(x, w_ref[...], preferred_element_type=jnp.float32)
    _write_h(h, al_ref, ar_ref, out_refs)


def _tc_mid(u, t, b, w, al, ar):
    din, dout = w.shape
    np_in = u.shape[0]
    out_specs, out_shape = _h_out_specs(dout)
    return pl.pallas_call(
        _tc_mid_body,
        grid=(_N // _BN,),
        in_specs=[
            pl.BlockSpec((np_in, _NC, _BN, _DS), lambda i: (0, 0, i, 0)),
            pl.BlockSpec((_NC, _BN, 16), lambda i: (0, i, 0)),
            pl.BlockSpec((din,), lambda i: (0,)),
            pl.BlockSpec((din, dout), lambda i: (0, 0)),
            pl.BlockSpec((dout,), lambda i: (0,)),
            pl.BlockSpec((dout,), lambda i: (0,)),
        ],
        out_specs=out_specs,
        out_shape=out_shape,
    )(u, t, b, w, al, ar)


def _tc_out_body(u_ref, t_ref, b_ref, o_ref):
    z = _norm(u_ref, t_ref, b_ref)
    m = jnp.max(z, axis=1, keepdims=True)
    ez = jnp.exp(z - m)
    o_ref[...] = z - m - jnp.log(jnp.sum(ez, axis=1, keepdims=True))


def _tc_out(u, t, b):
    dout = b.shape[0]
    np_in = u.shape[0]
    return pl.pallas_call(
        _tc_out_body,
        grid=(_N // _BN,),
        in_specs=[
            pl.BlockSpec((np_in, _NC, _BN, _DS), lambda i: (0, 0, i, 0)),
            pl.BlockSpec((_NC, _BN, 16), lambda i: (0, i, 0)),
            pl.BlockSpec((dout,), lambda i: (0,)),
        ],
        out_specs=pl.BlockSpec((_BN, dout), lambda i: (i, 0)),
        out_shape=jax.ShapeDtypeStruct((_N, dout), jnp.float32),
    )(u, t, b)


def kernel(feat, edge_index, W1, al1, ar1, b1, W2, al2, ar2, b2):
    src = edge_index[0].astype(jnp.int32).reshape(_NW, _NCH, _C)
    dst = edge_index[1].astype(jnp.int32).reshape(_NW, _NCH, _C)

    h1a, h1b, el1, er1 = _tc_project(feat, W1, al1, ar1)
    u1, t1 = _sc_layer1(h1a, h1b, el1.reshape(_N), er1.reshape(_N), src, dst)
    h2, el2, er2 = _tc_mid(u1, t1, b1, W2, al2, ar2)
    u2, t2 = _sc_layer2(h2, el2.reshape(_N), er2.reshape(_N), src, dst)
    return _tc_out(u2, t2, b2)


# speedup vs baseline: 1.1514x; 1.1514x over previous
import functools

import jax
import jax.numpy as jnp
from jax import lax
from jax.experimental import pallas as pl
from jax.experimental.pallas import tpu as pltpu
from jax.experimental.pallas import tpu_sc as plsc

_N = 10000
_E = 320000
_NC = 2
_NS = 16
_NW = _NC * _NS
_EW = _E // _NW
_C = 80
_NCH = _EW // _C
_G = _C // 16
_RPT = _N // _NS
_BN = 1000


def _mesh():
    return plsc.VectorSubcoreMesh(
        core_axis_name="c", subcore_axis_name="s",
        num_cores=_NC, num_subcores=_NS)


_DS = 64


def _make_sc_layer(NP):
    @functools.partial(
        pl.kernel,
        out_type=(
            jax.ShapeDtypeStruct((NP, _NC, _N, _DS), jnp.float32),
            jax.ShapeDtypeStruct((_NC, _N, 16), jnp.float32),
        ),
        mesh=_mesh(),
        compiler_params=pltpu.CompilerParams(
            needs_layout_passes=False, use_tc_tiling_on_sc=False),
        scratch_types=[
            pltpu.VMEM((_NCH, _C), jnp.int32),
            pltpu.VMEM((_NCH, _C), jnp.int32),
            pltpu.VMEM((_N,), jnp.float32),
            pltpu.VMEM((_N,), jnp.float32),
            pltpu.VMEM((_NCH, _C), jnp.float32),
        ] + [pltpu.VMEM((_C, _DS), jnp.float32)] * 3
          + [pltpu.VMEM((_C, _DS), jnp.float32)] * 2
          + [pltpu.VMEM((_C, 16), jnp.float32)] * 2
          + [
            pltpu.VMEM_SHARED((_N, _DS), jnp.float32),
            pltpu.VMEM_SHARED((_N, 16), jnp.float32),
        ] + [pltpu.SemaphoreType.DMA] * 12,
    )
    def sc_layer(*refs):
        h_hbms = refs[:NP]
        (el_hbm, er_hbm, src_hbm, dst_hbm, u_hbm, t_hbm,
         src_v, dst_v, el_v, er_v, ee_v) = refs[NP:NP + 11]
        hbf_bufs = list(refs[NP + 11:NP + 14])
        rows_bufs = list(refs[NP + 14:NP + 16])
        eec_bufs = list(refs[NP + 16:NP + 18])
        acc_sh, den_sh = refs[NP + 18:NP + 20]
        sems = refs[NP + 20:]
        sg = sems[0:3]
        ss = sems[3:5]
        se = sems[5:7]
        rows_v, eec_v = rows_bufs[0], eec_bufs[0]
        cid = lax.axis_index("c")
        sid = lax.axis_index("s")
        wid = sid * _NC + cid

        pltpu.sync_copy(src_hbm.at[wid], src_v)
        pltpu.sync_copy(dst_hbm.at[wid], dst_v)
        pltpu.sync_copy(el_hbm, el_v)
        pltpu.sync_copy(er_hbm, er_v)

        zero16 = jnp.zeros((16,), jnp.float32)

        def zero_rows(i, _):
            r = i // (_DS // 16)
            g = i - r * (_DS // 16)
            rows_v[r, pl.ds(pl.multiple_of(g * 16, 16), 16)] = zero16
            return 0
        lax.fori_loop(0, _C * _DS // 16, zero_rows, 0)

        def zero_eec(r, _):
            eec_v[r, :] = zero16
            return 0
        lax.fori_loop(0, _C, zero_eec, 0)

        def owned_chunks(fn):
            for k in range(-(-_NCH // _NS)):
                c = sid + _NS * k

                @pl.when(c < _NCH)
                def _run(c=c):
                    fn(pl.ds(pl.multiple_of(c * _C, _C), _C))

        def zero_acc(off):
            pltpu.sync_copy(rows_v, acc_sh.at[off])

        owned_chunks(zero_acc)
        owned_chunks(lambda off: pltpu.sync_copy(eec_v, den_sh.at[off]))
        plsc.subcore_barrier()

        def edge_body(c, _):
            for g in range(_G):
                off = pl.ds(g * 16, 16)
                s16 = src_v[c, off]
                d16 = dst_v[c, off]
                el16 = plsc.load_gather(el_v, [s16])
                er16 = plsc.load_gather(er_v, [d16])
                e = el16 + er16
                e = jnp.where(e > 0.0, e, 0.2 * e)
                ee_v[c, off] = jnp.exp(e)
            return 0
        lax.fori_loop(0, _NCH, edge_body, 0)

        lane0 = lax.iota(jnp.int32, 16) == 0
        _ROUNDS = _NCH // 6

        for p in range(NP):
            h_hbm = h_hbms[p]

            def scale(hbf, rows, eec, c, p=p):
                fc = jnp.full((16,), c, jnp.int32)

                def scale_rows4(q, _):
                    r0 = q * 4
                    for j in range(4):
                        r = r0 + j
                        b = plsc.load_gather(
                            ee_v, [fc, jnp.full((16,), r, jnp.int32)])
                        if p == 0:
                            eec[r, :] = jnp.where(lane0, b, 0.0)
                        for dg in range(_DS // 16):
                            off = pl.ds(dg * 16, 16)
                            rows[r, off] = hbf[r, off] * b
                    return 0
                lax.fori_loop(0, _C // 4, scale_rows4, 0)

            def gather_start(c, hbf, sgj, h_hbm=h_hbm):
                pltpu.async_copy(h_hbm.at[src_v.at[c]], hbf, sgj)

            def gather_wait(c, hbf, sgj, h_hbm=h_hbm):
                pltpu.make_async_copy(h_hbm.at[src_v.at[c]], hbf, sgj).wait()

            def scatter_start(c, rows, eec, ss, se, p=p):
                pltpu.async_copy(rows, acc_sh.at[dst_v.at[c]], ss, add=True)
                if p == 0:
                    pltpu.async_copy(eec, den_sh.at[dst_v.at[c]], se,
                                     add=True)

            def scatter_wait(c, rows, eec, ss, se, p=p):
                pltpu.make_async_copy(rows, acc_sh.at[dst_v.at[c]], ss).wait()
                if p == 0:
                    pltpu.make_async_copy(eec, den_sh.at[dst_v.at[c]],
                                          se).wait()

            for j in range(2):
                gather_start(j, hbf_bufs[j], sg[j])

            def chunk_step(c, j, k=None):
                jg = j % 3
                jn = (j + 2) % 3
                jr = j % 2
                gather_wait(c, hbf_bufs[jg], sg[jg])
                if isinstance(c, int):
                    if c + 2 < _NCH:
                        gather_start(c + 2, hbf_bufs[jn], sg[jn])
                else:
                    gather_start(c + 2, hbf_bufs[jn], sg[jn])
                if k is not None and j < 2:
                    @pl.when(k > 0)
                    def _wait_prev():
                        scatter_wait(c - 2, rows_bufs[jr], eec_bufs[jr],
                                     ss[jr], se[jr])
                else:
                    scatter_wait(c - 2, rows_bufs[jr], eec_bufs[jr],
                                 ss[jr], se[jr])
                scale(hbf_bufs[jg], rows_bufs[jr], eec_bufs[jr], c)
                scatter_start(c, rows_bufs[jr], eec_bufs[jr], ss[jr], se[jr])

            def ring_body(k, _):
                for j in range(6):
                    chunk_step(6 * k + j, j, k=k)
                return 0
            lax.fori_loop(0, _ROUNDS, ring_body, 0)

            for c in range(6 * _ROUNDS, _NCH):
                chunk_step(c, c - 6 * _ROUNDS + 6)
            for d in range(_NCH - 2, _NCH):
                jd = d % 2
                scatter_wait(d, rows_bufs[jd], eec_bufs[jd], ss[jd], se[jd])

            plsc.subcore_barrier()

            owned_chunks(
                lambda off, p=p: pltpu.sync_copy(acc_sh.at[off],
                                                 u_hbm.at[p, cid, off]))
            if p == 0:
                owned_chunks(
                    lambda off: pltpu.sync_copy(den_sh.at[off],
                                                t_hbm.at[cid, off]))
            if p + 1 < NP:
                lax.fori_loop(0, _C * _DS // 16, zero_rows, 0)
                owned_chunks(zero_acc)
                plsc.subcore_barrier()

    return sc_layer


_sc_layer1 = _make_sc_layer(2)
_sc_layer2 = _make_sc_layer(1)


def _h_out_specs(dout):
    np_ = dout // _DS
    specs = [pl.BlockSpec((_BN, _DS), lambda i: (i, 0))] * np_
    specs += [pl.BlockSpec((_BN, 1), lambda i: (i, 0))] * 2
    shapes = [jax.ShapeDtypeStruct((_N, _DS), jnp.float32)] * np_
    shapes += [jax.ShapeDtypeStruct((_N, 1), jnp.float32)] * 2
    return specs, shapes


def _write_h(h, al_ref, ar_ref, out_refs):
    np_ = len(out_refs) - 2
    for p in range(np_):
        out_refs[p][...] = h[:, p * _DS:(p + 1) * _DS]
    out_refs[np_][...] = jnp.dot(h, al_ref[...],
                                 preferred_element_type=jnp.float32)[:, None]
    out_refs[np_ + 1][...] = jnp.dot(h, ar_ref[...],
                                     preferred_element_type=jnp.float32)[:, None]


def _tc_project_body(x_ref, w_ref, al_ref, ar_ref, *out_refs):
    h = jnp.dot(x_ref[...], w_ref[...], preferred_element_type=jnp.float32)
    _write_h(h, al_ref, ar_ref, out_refs)


def _tc_project(x, w, al, ar):
    din, dout = w.shape
    out_specs, out_shape = _h_out_specs(dout)
    return pl.pallas_call(
        _tc_project_body,
        grid=(_N // _BN,),
        in_specs=[
            pl.BlockSpec((_BN, din), lambda i: (i, 0)),
            pl.BlockSpec((din, dout), lambda i: (0, 0)),
            pl.BlockSpec((dout,), lambda i: (0,)),
            pl.BlockSpec((dout,), lambda i: (0,)),
        ],
        out_specs=out_specs,
        out_shape=out_shape,
    )(x, w, al, ar)


def _norm(u_ref, t_ref, b_ref):
    np_ = u_ref.shape[0]
    u = jnp.concatenate(
        [u_ref[p, 0] + u_ref[p, 1] for p in range(np_)], axis=1)
    s = t_ref[0, :, 0:1] + t_ref[1, :, 0:1]
    s = jnp.where(s == 0.0, 1.0, s)
    return u / s + b_ref[...][None, :]


def _tc_mid_body(u_ref, t_ref, b_ref, w_ref, al_ref, ar_ref, *out_refs):
    x = jnp.maximum(_norm(u_ref, t_ref, b_ref), 0.0)
    h = jnp.dot(x, w_ref[...], preferred_element_type=jnp.float32)
    _write_h(h, al_ref, ar_ref, out_refs)


def _tc_mid(u, t, b, w, al, ar):
    din, dout = w.shape
    np_in = u.shape[0]
    out_specs, out_shape = _h_out_specs(dout)
    return pl.pallas_call(
        _tc_mid_body,
        grid=(_N // _BN,),
        in_specs=[
            pl.BlockSpec((np_in, _NC, _BN, _DS), lambda i: (0, 0, i, 0)),
            pl.BlockSpec((_NC, _BN, 16), lambda i: (0, i, 0)),
            pl.BlockSpec((din,), lambda i: (0,)),
            pl.BlockSpec((din, dout), lambda i: (0, 0)),
            pl.BlockSpec((dout,), lambda i: (0,)),
            pl.BlockSpec((dout,), lambda i: (0,)),
        ],
        out_specs=out_specs,
        out_shape=out_shape,
    )(u, t, b, w, al, ar)


def _tc_out_body(u_ref, t_ref, b_ref, o_ref):
    z = _norm(u_ref, t_ref, b_ref)
    m = jnp.max(z, axis=1, keepdims=True)
    ez = jnp.exp(z - m)
    o_ref[...] = z - m - jnp.log(jnp.sum(ez, axis=1, keepdims=True))


def _tc_out(u, t, b):
    dout = b.shape[0]
    np_in = u.shape[0]
    return pl.pallas_call(
        _tc_out_body,
        grid=(_N // _BN,),
        in_specs=[
            pl.BlockSpec((np_in, _NC, _BN, _DS), lambda i: (0, 0, i, 0)),
            pl.BlockSpec((_NC, _BN, 16), lambda i: (0, i, 0)),
            pl.BlockSpec((dout,), lambda i: (0,)),
        ],
        out_specs=pl.BlockSpec((_BN, dout), lambda i: (i, 0)),
        out_shape=jax.ShapeDtypeStruct((_N, dout), jnp.float32),
    )(u, t, b)


def kernel(feat, edge_index, W1, al1, ar1, b1, W2, al2, ar2, b2):
    src = edge_index[0].astype(jnp.int32).reshape(_NW, _NCH, _C)
    dst = edge_index[1].astype(jnp.int32).reshape(_NW, _NCH, _C)

    h1a, h1b, el1, er1 = _tc_project(feat, W1, al1, ar1)
    u1, t1 = _sc_layer1(h1a, h1b, el1.reshape(_N), er1.reshape(_N), src, dst)
    h2, el2, er2 = _tc_mid(u1, t1, b1, W2, al2, ar2)
    u2, t2 = _sc_layer2(h2, el2.reshape(_N), er2.reshape(_N), src, dst)
    return _tc_out(u2, t2, b2)


# speedup vs baseline: 1.9539x; 1.6969x over previous
import functools

import jax
import jax.numpy as jnp
from jax import lax
from jax.experimental import pallas as pl
from jax.experimental.pallas import tpu as pltpu
from jax.experimental.pallas import tpu_sc as plsc

_N = 10000
_E = 320000
_NC = 2
_NS = 16
_NW = _NC * _NS
_EW = _E // _NW
_C = 80
_NCH = _EW // _C
_G = _C // 16
_RPT = _N // _NS
_BN = 1000


def _mesh():
    return plsc.VectorSubcoreMesh(
        core_axis_name="c", subcore_axis_name="s",
        num_cores=_NC, num_subcores=_NS)


_DS = 64


def _make_sc_layer(NP):
    @functools.partial(
        pl.kernel,
        out_type=(
            jax.ShapeDtypeStruct((NP, _NC, _N, _DS), jnp.float32),
            jax.ShapeDtypeStruct((_NC, _N, 16), jnp.float32),
        ),
        mesh=_mesh(),
        compiler_params=pltpu.CompilerParams(
            needs_layout_passes=False, use_tc_tiling_on_sc=False),
        scratch_types=[
            pltpu.VMEM((_NCH, _C), jnp.int32),
            pltpu.VMEM((_NCH, _C), jnp.int32),
            pltpu.VMEM((_N,), jnp.float32),
            pltpu.VMEM((_N,), jnp.float32),
            pltpu.VMEM((_NCH, _C), jnp.float32),
            pltpu.VMEM((4, _C, _DS), jnp.float32),
            pltpu.VMEM((4, _C, 16), jnp.float32),
            pltpu.VMEM_SHARED((_N, _DS), jnp.float32),
            pltpu.VMEM_SHARED((_N, 16), jnp.float32),
        ] + [pltpu.SemaphoreType.DMA] * 12,
    )
    def sc_layer(*refs):
        h_hbms = refs[:NP]
        (el_hbm, er_hbm, src_hbm, dst_hbm, u_hbm, t_hbm,
         src_v, dst_v, el_v, er_v, ee_v, rows_ring, eec_ring,
         acc_sh, den_sh, *sems) = refs[NP:]
        sg = sems[0:4]
        ss = sems[4:8]
        se = sems[8:12]
        rows_bufs = [rows_ring.at[j] for j in range(4)]
        eec_bufs = [eec_ring.at[j] for j in range(4)]
        rows_v, eec_v = rows_bufs[0], eec_bufs[0]
        cid = lax.axis_index("c")
        sid = lax.axis_index("s")
        wid = sid * _NC + cid

        pltpu.sync_copy(src_hbm.at[wid], src_v)
        pltpu.sync_copy(dst_hbm.at[wid], dst_v)
        pltpu.sync_copy(el_hbm, el_v)
        pltpu.sync_copy(er_hbm, er_v)

        zero16 = jnp.zeros((16,), jnp.float32)

        def zero_rows(i, _):
            r = i // (_DS // 16)
            g = i - r * (_DS // 16)
            rows_v[r, pl.ds(pl.multiple_of(g * 16, 16), 16)] = zero16
            return 0
        lax.fori_loop(0, _C * _DS // 16, zero_rows, 0)

        def zero_eec(r, _):
            eec_v[r, :] = zero16
            return 0
        lax.fori_loop(0, _C, zero_eec, 0)

        def owned_chunks(fn):
            for k in range(-(-_NCH // _NS)):
                c = sid + _NS * k

                @pl.when(c < _NCH)
                def _run(c=c):
                    fn(pl.ds(pl.multiple_of(c * _C, _C), _C))

        def zero_acc(off):
            pltpu.sync_copy(rows_v, acc_sh.at[off])

        owned_chunks(zero_acc)
        owned_chunks(lambda off: pltpu.sync_copy(eec_v, den_sh.at[off]))
        plsc.subcore_barrier()

        def edge_body(c, _):
            for g in range(_G):
                off = pl.ds(g * 16, 16)
                s16 = src_v[c, off]
                d16 = dst_v[c, off]
                el16 = plsc.load_gather(el_v, [s16])
                er16 = plsc.load_gather(er_v, [d16])
                e = el16 + er16
                e = jnp.where(e > 0.0, e, 0.2 * e)
                ee_v[c, off] = jnp.exp(e)
            return 0
        lax.fori_loop(0, _NCH, edge_body, 0)

        lane0 = lax.iota(jnp.int32, 16) == 0
        _ROUNDS = _NCH // 4

        for p in range(NP):
            h_hbm = h_hbms[p]

            def scale(rows, eec, c, p=p):
                fc = jnp.full((16,), c, jnp.int32)

                def scale_rows4(q, _):
                    r0 = q * 4
                    for j in range(4):
                        r = r0 + j
                        b = plsc.load_gather(
                            ee_v, [fc, jnp.full((16,), r, jnp.int32)])
                        if p == 0:
                            eec[r, :] = jnp.where(lane0, b, 0.0)
                        for dg in range(_DS // 16):
                            off = pl.ds(dg * 16, 16)
                            rows[r, off] = rows[r, off] * b
                    return 0
                lax.fori_loop(0, _C // 4, scale_rows4, 0)

            def gather_start(c, rows, sg, h_hbm=h_hbm):
                pltpu.async_copy(h_hbm.at[src_v.at[c]], rows, sg)

            def gather_wait(c, rows, sg, h_hbm=h_hbm):
                pltpu.make_async_copy(h_hbm.at[src_v.at[c]], rows, sg).wait()

            def scatter_start(c, rows, eec, ss, se, p=p):
                pltpu.async_copy(rows, acc_sh.at[dst_v.at[c]], ss, add=True)
                if p == 0:
                    pltpu.async_copy(eec, den_sh.at[dst_v.at[c]], se,
                                     add=True)

            def scatter_wait(c, rows, eec, ss, se, p=p):
                pltpu.make_async_copy(rows, acc_sh.at[dst_v.at[c]], ss).wait()
                if p == 0:
                    pltpu.make_async_copy(eec, den_sh.at[dst_v.at[c]],
                                          se).wait()

            for j in range(3):
                gather_start(j, rows_bufs[j], sg[j])

            def ring_body(k, _):
                for j in range(4):
                    c = 4 * k + j
                    jm1 = (j - 1) % 4
                    jp3 = (j + 3) % 4
                    gather_wait(c, rows_bufs[j], sg[j])
                    scale(rows_bufs[j], eec_bufs[j], c)
                    scatter_start(c, rows_bufs[j], eec_bufs[j], ss[j], se[j])
                    if j == 0:
                        @pl.when(k > 0)
                        def _wait_prev():
                            scatter_wait(c - 1, rows_bufs[jm1],
                                         eec_bufs[jm1], ss[jm1], se[jm1])
                    else:
                        scatter_wait(c - 1, rows_bufs[jm1], eec_bufs[jm1],
                                     ss[jm1], se[jm1])

                    @pl.when(c + 3 < _NCH)
                    def _next_gather():
                        gather_start(c + 3, rows_bufs[jp3], sg[jp3])
                return 0
            lax.fori_loop(0, _ROUNDS, ring_body, 0)

            last = _NCH - 1
            jl = last % 4
            gather_wait(last, rows_bufs[jl], sg[jl])
            scale(rows_bufs[jl], eec_bufs[jl], last)
            scatter_start(last, rows_bufs[jl], eec_bufs[jl], ss[jl], se[jl])
            scatter_wait(last - 1, rows_bufs[(jl - 1) % 4],
                         eec_bufs[(jl - 1) % 4], ss[(jl - 1) % 4],
                         se[(jl - 1) % 4])
            scatter_wait(last, rows_bufs[jl], eec_bufs[jl], ss[jl], se[jl])

            plsc.subcore_barrier()

            owned_chunks(
                lambda off, p=p: pltpu.sync_copy(acc_sh.at[off],
                                                 u_hbm.at[p, cid, off]))
            if p == 0:
                owned_chunks(
                    lambda off: pltpu.sync_copy(den_sh.at[off],
                                                t_hbm.at[cid, off]))
            if p + 1 < NP:
                lax.fori_loop(0, _C * _DS // 16, zero_rows, 0)
                owned_chunks(zero_acc)
                plsc.subcore_barrier()

    return sc_layer


_sc_layer1 = _make_sc_layer(2)
_sc_layer2 = _make_sc_layer(1)


def _h_out_specs(dout):
    np_ = dout // _DS
    specs = [pl.BlockSpec((_BN, _DS), lambda i: (i, 0))] * np_
    specs += [pl.BlockSpec((_BN, 1), lambda i: (i, 0))] * 2
    shapes = [jax.ShapeDtypeStruct((_N, _DS), jnp.float32)] * np_
    shapes += [jax.ShapeDtypeStruct((_N, 1), jnp.float32)] * 2
    return specs, shapes


def _write_h(h, al_ref, ar_ref, out_refs):
    np_ = len(out_refs) - 2
    for p in range(np_):
        out_refs[p][...] = h[:, p * _DS:(p + 1) * _DS]
    out_refs[np_][...] = jnp.dot(h, al_ref[...],
                                 preferred_element_type=jnp.float32)[:, None]
    out_refs[np_ + 1][...] = jnp.dot(h, ar_ref[...],
                                     preferred_element_type=jnp.float32)[:, None]


def _tc_project_body(x_ref, w_ref, al_ref, ar_ref, *out_refs):
    h = jnp.dot(x_ref[...], w_ref[...], preferred_element_type=jnp.float32)
    _write_h(h, al_ref, ar_ref, out_refs)


def _tc_project(x, w, al, ar):
    din, dout = w.shape
    out_specs, out_shape = _h_out_specs(dout)
    return pl.pallas_call(
        _tc_project_body,
        grid=(_N // _BN,),
        in_specs=[
            pl.BlockSpec((_BN, din), lambda i: (i, 0)),
            pl.BlockSpec((din, dout), lambda i: (0, 0)),
            pl.BlockSpec((dout,), lambda i: (0,)),
            pl.BlockSpec((dout,), lambda i: (0,)),
        ],
        out_specs=out_specs,
        out_shape=out_shape,
    )(x, w, al, ar)


def _norm(u_ref, t_ref, b_ref):
    np_ = u_ref.shape[0]
    u = jnp.concatenate(
        [u_ref[p, 0] + u_ref[p, 1] for p in range(np_)], axis=1)
    s = t_ref[0, :, 0:1] + t_ref[1, :, 0:1]
    s = jnp.where(s == 0.0, 1.0, s)
    return u / s + b_ref[...][None, :]


def _tc_mid_body(u_ref, t_ref, b_ref, w_ref, al_ref, ar_ref, *out_refs):
    x = jnp.maximum(_norm(u_ref, t_ref, b_ref), 0.0)
    h = jnp.dot(x, w_ref[...], preferred_element_type=jnp.float32)
    _write_h(h, al_ref, ar_ref, out_refs)


def _tc_mid(u, t, b, w, al, ar):
    din, dout = w.shape
    np_in = u.shape[0]
    out_specs, out_shape = _h_out_specs(dout)
    return pl.pallas_call(
        _tc_mid_body,
        grid=(_N // _BN,),
        in_specs=[
            pl.BlockSpec((np_in, _NC, _BN, _DS), lambda i: (0, 0, i, 0)),
            pl.BlockSpec((_NC, _BN, 16), lambda i: (0, i, 0)),
            pl.BlockSpec((din,), lambda i: (0,)),
            pl.BlockSpec((din, dout), lambda i: (0, 0)),
            pl.BlockSpec((dout,), lambda i: (0,)),
            pl.BlockSpec((dout,), lambda i: (0,)),
        ],
        out_specs=out_specs,
        out_shape=out_shape,
    )(u, t, b, w, al, ar)


def _tc_out_body(u_ref, t_ref, b_ref, o_ref):
    z = _norm(u_ref, t_ref, b_ref)
    m = jnp.max(z, axis=1, keepdims=True)
    ez = jnp.exp(z - m)
    o_ref[...] = z - m - jnp.log(jnp.sum(ez, axis=1, keepdims=True))


def _tc_out(u, t, b):
    dout = b.shape[0]
    np_in = u.shape[0]
    return pl.pallas_call(
        _tc_out_body,
        grid=(_N // _BN,),
        in_specs=[
            pl.BlockSpec((np_in, _NC, _BN, _DS), lambda i: (0, 0, i, 0)),
            pl.BlockSpec((_NC, _BN, 16), lambda i: (0, i, 0)),
            pl.BlockSpec((dout,), lambda i: (0,)),
        ],
        out_specs=pl.BlockSpec((_BN, dout), lambda i: (i, 0)),
        out_shape=jax.ShapeDtypeStruct((_N, dout), jnp.float32),
    )(u, t, b)


def kernel(feat, edge_index, W1, al1, ar1, b1, W2, al2, ar2, b2):
    src = edge_index[0].astype(jnp.int32).reshape(_NW, _NCH, _C)
    dst = edge_index[1].astype(jnp.int32).reshape(_NW, _NCH, _C)

    h1a, h1b, el1, er1 = _tc_project(feat, W1, al1, ar1)
    u1, t1 = _sc_layer1(h1a, h1b, el1.reshape(_N), er1.reshape(_N), src, dst)
    h2, el2, er2 = _tc_mid(u1, t1, b1, W2, al2, ar2)
    u2, t2 = _sc_layer2(h2, el2.reshape(_N), er2.reshape(_N), src, dst)
    return _tc_out(u2, t2, b2)
